# baseline (speedup 1.0000x reference)
import math

import jax
import jax.numpy as jnp
from jax import lax
from jax.experimental import pallas as pl

N_LM = 800
N_TG = 224
N = N_LM + N_TG
DIM_IN = 128
DIM_AS = 64
DIM_Z = 16
DIM_INNER = 256
DIM_OUT = 2
N_HEAD = 4
K_TOP = math.ceil(N_LM * 0.6)
LAMBDA_1 = 0.8
LAMBDA_2 = 0.4
_INV_SQRT_DAS = 1.0 / math.sqrt(DIM_AS)


def _softplus(x):
    return jnp.maximum(x, 0.0) + jnp.log1p(jnp.exp(-jnp.abs(x)))


def _body(x_ref, logd_ref, yc_ref, eps_ref,
          disw_ref, disb_ref, kw_ref, kb_ref,
          wq_ref, bq_ref, wk_ref, bk_ref,
          e1w_ref, e1b_ref, e2w_ref, e2b_ref, dw_ref, db_ref,
          g1x_ref, g1y_ref, g1b_ref, g2w_ref, g2b_ref,
          muw_ref, mub_ref, vaw_ref, vab_ref,
          ua_ref, ub_ref, urb_ref,
          c1w_ref, c1b_ref, c2w_ref, c2b_ref,
          p1a_ref, p1b_ref, p1bias_ref, p2w_ref, p2b_ref,
          y_ref, loss_ref):
    f32 = jnp.float32
    X = x_ref[:]
    logd = logd_ref[:]

    coe = jnp.dot(X, disw_ref[:], preferred_element_type=f32) + disb_ref[:]
    dist = coe[:, 0:1] * logd + coe[:, 1:2] + coe[:, 2:3]
    kappa = jnp.maximum(
        jnp.dot(X, kw_ref[:], preferred_element_type=f32) + kb_ref[:], 0.0)

    Q = jnp.dot(X, wq_ref[:], preferred_element_type=f32) + bq_ref[:]
    K = jnp.dot(X, wk_ref[:], preferred_element_type=f32) + bk_ref[:]
    edge_as = jnp.zeros((N, N), f32)
    for h in range(N_HEAD):
        qh = Q[:, h * DIM_AS:(h + 1) * DIM_AS].astype(jnp.bfloat16)
        kh = K[:, h * DIM_AS:(h + 1) * DIM_AS].astype(jnp.bfloat16)
        s = lax.dot_general(qh, kh, (((1,), (1,)), ((), ())),
                            preferred_element_type=f32) * _INV_SQRT_DAS
        s = s - jnp.max(s, axis=1, keepdims=True)
        e = jnp.exp(s)
        edge_as = edge_as + e / jnp.sum(e, axis=1, keepdims=True)
    edge_as = edge_as * (1.0 / N_HEAD)

    ri = lax.broadcasted_iota(jnp.int32, (N, N), 0)
    ci = lax.broadcasted_iota(jnp.int32, (N, N), 1)
    blocked = (ri >= N_LM) & (ci >= N_LM) & (ri != ci)
    adj0 = jnp.where(blocked, 0.0,
                     jnp.exp(-jnp.abs(dist - dist.T)) * kappa + edge_as)

    bits = lax.bitcast_convert_type(adj0, jnp.int32)
    kth = jnp.zeros((N, 1), jnp.int32)
    for b in range(30, -1, -1):
        cand = kth | (1 << b)
        cnt = jnp.sum((bits >= cand).astype(jnp.int32), axis=1, keepdims=True)
        kth = jnp.where(cnt >= K_TOP, cand, kth)
    keep = bits >= kth

    rowmax = jnp.max(adj0, axis=1, keepdims=True)
    ex = jnp.where(keep, jnp.exp(adj0 - rowmax), 0.0)
    A = ex / jnp.sum(ex, axis=1, keepdims=True)

    AX = jnp.dot(A, X, preferred_element_type=f32)
    h1 = jnp.maximum(jnp.dot(AX, e1w_ref[:], preferred_element_type=f32)
                     + e1b_ref[:], 0.0)
    Ah1 = jnp.dot(A, h1, preferred_element_type=f32)
    h_enc = jnp.dot(Ah1, e2w_ref[:], preferred_element_type=f32) + e2b_ref[:]
    x_dec = jnp.dot(h_enc, dw_ref[:], preferred_element_type=f32) + db_ref[:]
    diff = x_dec - X
    g_loss = jnp.sum(diff * diff) * (1.0 / (N * DIM_IN))

    Yc = yc_ref[:]
    AY = jnp.dot(A, Yc, preferred_element_type=f32)
    hz = jnp.maximum(jnp.dot(AX, g1x_ref[:], preferred_element_type=f32)
                     + jnp.dot(AY, g1y_ref[:], preferred_element_type=f32)
                     + g1b_ref[:], 0.0)
    Ahz = jnp.dot(A, hz, preferred_element_type=f32)
    h_enc_u = jnp.dot(Ahz, g2w_ref[:], preferred_element_type=f32) + g2b_ref[:]
    z_mu = jnp.dot(h_enc_u, muw_ref[:], preferred_element_type=f32) + mub_ref[:]
    z_sigma = _softplus(jnp.dot(h_enc_u, vaw_ref[:], preferred_element_type=f32)
                        + vab_ref[:]) + 1e-10
    z0 = z_mu + z_sigma * eps_ref[:]

    W1 = c1w_ref[:]
    W2 = c2w_ref[:]
    m21 = jnp.dot(W2, W1, preferred_element_type=f32)
    di = lax.broadcasted_iota(jnp.int32, (DIM_Z, DIM_Z), 0)
    dj = lax.broadcasted_iota(jnp.int32, (DIM_Z, DIM_Z), 1)
    coef = jnp.sum(jnp.where(di == dj, m21, 0.0), axis=1, keepdims=True)
    z = z0
    logp = jnp.zeros((N, 1), f32)
    dt = 0.25
    for _ in range(4):
        h = jnp.tanh(jnp.dot(z, W1, preferred_element_type=f32) + c1b_ref[:])
        f = jnp.dot(h, W2, preferred_element_type=f32) + c2b_ref[:]
        tr = jnp.dot(1.0 - h * h, coef, preferred_element_type=f32)
        z = z + dt * f
        logp = logp - dt * tr
    z_t = z

    u = jnp.dot(z_t, ua_ref[:], preferred_element_type=f32)
    v = jnp.dot(z_t, ub_ref[:], preferred_element_type=f32)
    logits = v + u.T + urb_ref[0, 0]
    rec_loss = jnp.sum(jnp.where(keep, _softplus(logits), 0.0)) * (1.0 / (N * N))
    elbo = (rec_loss - 0.5 * jnp.mean(z_t * z_t)
            + 0.5 * jnp.mean(z0 * z0) + jnp.mean(logp))

    hc1 = jnp.maximum(jnp.dot(h_enc_u, p1a_ref[:], preferred_element_type=f32)
                      + jnp.dot(z_t, p1b_ref[:], preferred_element_type=f32)
                      + p1bias_ref[:], 0.0)
    lm_mean = jnp.sum(Yc, axis=0, keepdims=True) * (1.0 / N_LM)
    y = jnp.dot(hc1, p2w_ref[:], preferred_element_type=f32) + p2b_ref[:] + lm_mean
    y_ref[:] = y[N_LM:, :]
    loss_ref[:] = jnp.reshape(g_loss * LAMBDA_1 - elbo * LAMBDA_2, (1, 1))


def kernel(lm_X, lm_Y, tg_X, tg_Y, lm_delay, tg_delay, params):
    p = params
    X = jnp.concatenate((lm_X, tg_X), axis=0)
    logd = jnp.concatenate((lm_delay, tg_delay), axis=0)[:, None]
    Yc = jnp.concatenate((lm_Y, jnp.zeros_like(tg_Y)), axis=0)
    eps = jax.random.normal(jax.random.key(42), (N, DIM_Z), dtype=jnp.float32)

    def r(x):
        return x[None, :] if x.ndim == 1 else x

    operands = (
        X, logd, Yc, eps,
        p['dis_co_W'], r(p['dis_co_b']), p['kappa_W'], r(p['kappa_b']),
        p['wq'], r(p['bq']), p['wk'], r(p['bk']),
        p['enc1_W'], r(p['enc1_b']), p['enc2_W'], r(p['enc2_b']),
        p['dec_W'], r(p['dec_b']),
        p['gnn1_W'][:DIM_IN], p['gnn1_W'][DIM_IN:], r(p['gnn1_b']),
        p['gnn2_W'], r(p['gnn2_b']),
        p['mu_W'], r(p['mu_b']), p['var_W'], r(p['var_b']),
        p['adj_rec_W'][:DIM_Z], p['adj_rec_W'][DIM_Z:], r(p['adj_rec_b']),
        p['cnf_W1'], r(p['cnf_b1']), p['cnf_W2'], r(p['cnf_b2']),
        p['pred1_W'][:DIM_Z], p['pred1_W'][DIM_Z:], r(p['pred1_b']),
        p['pred2_W'], r(p['pred2_b']),
    )
    y, loss = pl.pallas_call(
        _body,
        out_shape=(
            jax.ShapeDtypeStruct((N_TG, DIM_OUT), jnp.float32),
            jax.ShapeDtypeStruct((1, 1), jnp.float32),
        ),
    )(*operands)
    return y, loss[0, 0]


# speedup vs baseline: 1.0650x; 1.0650x over previous
import math

import jax
import jax.numpy as jnp
from jax import lax
from jax.experimental import pallas as pl

N_LM = 800
N_TG = 224
N = N_LM + N_TG
DIM_IN = 128
DIM_AS = 64
DIM_Z = 16
DIM_INNER = 256
DIM_OUT = 2
N_HEAD = 4
K_TOP = math.ceil(N_LM * 0.6)
LAMBDA_1 = 0.8
LAMBDA_2 = 0.4
_INV_SQRT_DAS = 1.0 / math.sqrt(DIM_AS)


def _softplus(x):
    return jnp.maximum(x, 0.0) + jnp.log1p(jnp.exp(-jnp.abs(x)))


def _body(x_ref, logd_ref, yc_ref, eps_ref,
          disw_ref, disb_ref, kw_ref, kb_ref,
          wq_ref, bq_ref, wk_ref, bk_ref,
          e1w_ref, e1b_ref, e2w_ref, e2b_ref, dw_ref, db_ref,
          g1x_ref, g1y_ref, g1b_ref, g2w_ref, g2b_ref,
          muw_ref, mub_ref, vaw_ref, vab_ref,
          ua_ref, ub_ref, urb_ref,
          c1w_ref, c1b_ref, c2w_ref, c2b_ref,
          p1a_ref, p1b_ref, p1bias_ref, p2w_ref, p2b_ref,
          y_ref, loss_ref):
    f32 = jnp.float32
    X = x_ref[:]
    logd = logd_ref[:]

    coe = jnp.dot(X, disw_ref[:], preferred_element_type=f32) + disb_ref[:]
    dist = coe[:, 0:1] * logd + coe[:, 1:2] + coe[:, 2:3]
    kappa = jnp.maximum(
        jnp.dot(X, kw_ref[:], preferred_element_type=f32) + kb_ref[:], 0.0)

    Q = jnp.dot(X, wq_ref[:], preferred_element_type=f32) + bq_ref[:]
    K = jnp.dot(X, wk_ref[:], preferred_element_type=f32) + bk_ref[:]
    edge_as = jnp.zeros((N, N), f32)
    for h in range(N_HEAD):
        qh = Q[:, h * DIM_AS:(h + 1) * DIM_AS]
        kh = K[:, h * DIM_AS:(h + 1) * DIM_AS]
        s = lax.dot_general(qh, kh, (((1,), (1,)), ((), ())),
                            preferred_element_type=f32) * _INV_SQRT_DAS
        s = s - jnp.max(s, axis=1, keepdims=True)
        e = jnp.exp(s)
        edge_as = edge_as + e / jnp.sum(e, axis=1, keepdims=True)
    edge_as = edge_as * (1.0 / N_HEAD)

    ri = lax.broadcasted_iota(jnp.int32, (N, N), 0)
    ci = lax.broadcasted_iota(jnp.int32, (N, N), 1)
    blocked = (ri >= N_LM) & (ci >= N_LM) & (ri != ci)
    adj0 = jnp.where(blocked, 0.0,
                     jnp.exp(-jnp.abs(dist - dist.T)) * kappa + edge_as)

    bits = lax.bitcast_convert_type(adj0, jnp.int32)
    kth = jnp.zeros((N, 1), jnp.int32)
    for b in range(30, -1, -1):
        cand = kth | (1 << b)
        cnt = jnp.sum((bits >= cand).astype(jnp.int32), axis=1, keepdims=True)
        kth = jnp.where(cnt >= K_TOP, cand, kth)
    keep = bits >= kth

    rowmax = jnp.max(adj0, axis=1, keepdims=True)
    ex = jnp.where(keep, jnp.exp(adj0 - rowmax), 0.0)
    A = ex / jnp.sum(ex, axis=1, keepdims=True)

    AX = jnp.dot(A, X, preferred_element_type=f32)
    h1 = jnp.maximum(jnp.dot(AX, e1w_ref[:], preferred_element_type=f32)
                     + e1b_ref[:], 0.0)
    Ah1 = jnp.dot(A, h1, preferred_element_type=f32)
    h_enc = jnp.dot(Ah1, e2w_ref[:], preferred_element_type=f32) + e2b_ref[:]
    x_dec = jnp.dot(h_enc, dw_ref[:], preferred_element_type=f32) + db_ref[:]
    diff = x_dec - X
    g_loss = jnp.sum(diff * diff) * (1.0 / (N * DIM_IN))

    Yc = yc_ref[:]
    AY = jnp.dot(A, Yc, preferred_element_type=f32)
    hz = jnp.maximum(jnp.dot(AX, g1x_ref[:], preferred_element_type=f32)
                     + jnp.dot(AY, g1y_ref[:], preferred_element_type=f32)
                     + g1b_ref[:], 0.0)
    Ahz = jnp.dot(A, hz, preferred_element_type=f32)
    h_enc_u = jnp.dot(Ahz, g2w_ref[:], preferred_element_type=f32) + g2b_ref[:]
    z_mu = jnp.dot(h_enc_u, muw_ref[:], preferred_element_type=f32) + mub_ref[:]
    z_sigma = _softplus(jnp.dot(h_enc_u, vaw_ref[:], preferred_element_type=f32)
                        + vab_ref[:]) + 1e-10
    z0 = z_mu + z_sigma * eps_ref[:]

    W1 = c1w_ref[:]
    W2 = c2w_ref[:]
    m21 = jnp.dot(W2, W1, preferred_element_type=f32)
    di = lax.broadcasted_iota(jnp.int32, (DIM_Z, DIM_Z), 0)
    dj = lax.broadcasted_iota(jnp.int32, (DIM_Z, DIM_Z), 1)
    coef = jnp.sum(jnp.where(di == dj, m21, 0.0), axis=1, keepdims=True)
    z = z0
    logp = jnp.zeros((N, 1), f32)
    dt = 0.25
    for _ in range(4):
        h = jnp.tanh(jnp.dot(z, W1, preferred_element_type=f32) + c1b_ref[:])
        f = jnp.dot(h, W2, preferred_element_type=f32) + c2b_ref[:]
        tr = jnp.dot(1.0 - h * h, coef, preferred_element_type=f32)
        z = z + dt * f
        logp = logp - dt * tr
    z_t = z

    u = jnp.dot(z_t, ua_ref[:], preferred_element_type=f32)
    v = jnp.dot(z_t, ub_ref[:], preferred_element_type=f32)
    logits = v + u.T + urb_ref[0, 0]
    rec_loss = jnp.sum(jnp.where(keep, _softplus(logits), 0.0)) * (1.0 / (N * N))
    elbo = (rec_loss - 0.5 * jnp.mean(z_t * z_t)
            + 0.5 * jnp.mean(z0 * z0) + jnp.mean(logp))

    hc1 = jnp.maximum(jnp.dot(h_enc_u, p1a_ref[:], preferred_element_type=f32)
                      + jnp.dot(z_t, p1b_ref[:], preferred_element_type=f32)
                      + p1bias_ref[:], 0.0)
    lm_mean = jnp.sum(Yc, axis=0, keepdims=True) * (1.0 / N_LM)
    y = jnp.dot(hc1, p2w_ref[:], preferred_element_type=f32) + p2b_ref[:] + lm_mean
    y_ref[:] = y[N_LM:, :]
    loss_ref[:] = jnp.reshape(g_loss * LAMBDA_1 - elbo * LAMBDA_2, (1, 1))


def kernel(lm_X, lm_Y, tg_X, tg_Y, lm_delay, tg_delay, params):
    p = params
    X = jnp.concatenate((lm_X, tg_X), axis=0)
    logd = jnp.concatenate((lm_delay, tg_delay), axis=0)[:, None]
    Yc = jnp.concatenate((lm_Y, jnp.zeros_like(tg_Y)), axis=0)
    eps = jax.random.normal(jax.random.key(42), (N, DIM_Z), dtype=jnp.float32)

    def r(x):
        return x[None, :] if x.ndim == 1 else x

    operands = (
        X, logd, Yc, eps,
        p['dis_co_W'], r(p['dis_co_b']), p['kappa_W'], r(p['kappa_b']),
        p['wq'], r(p['bq']), p['wk'], r(p['bk']),
        p['enc1_W'], r(p['enc1_b']), p['enc2_W'], r(p['enc2_b']),
        p['dec_W'], r(p['dec_b']),
        p['gnn1_W'][:DIM_IN], p['gnn1_W'][DIM_IN:], r(p['gnn1_b']),
        p['gnn2_W'], r(p['gnn2_b']),
        p['mu_W'], r(p['mu_b']), p['var_W'], r(p['var_b']),
        p['adj_rec_W'][:DIM_Z], p['adj_rec_W'][DIM_Z:], r(p['adj_rec_b']),
        p['cnf_W1'], r(p['cnf_b1']), p['cnf_W2'], r(p['cnf_b2']),
        p['pred1_W'][:DIM_Z], p['pred1_W'][DIM_Z:], r(p['pred1_b']),
        p['pred2_W'], r(p['pred2_b']),
    )
    y, loss = pl.pallas_call(
        _body,
        out_shape=(
            jax.ShapeDtypeStruct((N_TG, DIM_OUT), jnp.float32),
            jax.ShapeDtypeStruct((1, 1), jnp.float32),
        ),
    )(*operands)
    return y, loss[0, 0]
